# baseline (speedup 1.0000x reference)
import jax
import jax.numpy as jnp
from jax.experimental import pallas as pl

N = 100000
K = 2


def _lsm_body(x_ref, o_ref):
    x = x_ref[...]
    m = jnp.max(x, axis=1, keepdims=True)
    e = jnp.exp(x - m)
    s = jnp.sum(e, axis=1, keepdims=True)
    o_ref[...] = x - m - jnp.log(s)


def _seg_min(vals, idx, n):
    out = jax.ops.segment_min(vals, idx, num_segments=n)
    big = jnp.finfo(vals.dtype).max
    return jnp.where(out >= big, 0.0, out)


def _tag_conv(x, row, col, norm, Ws, b, n):
    out = x @ Ws[0]
    h = x
    for k in range(1, K + 1):
        m = h[row] * norm[:, None]
        h = _seg_min(m, col, n)
        out = out + h @ Ws[k]
    return out + b


def kernel(x, edge_index, W1_0, W1_1, W1_2, b1, W2_0, W2_1, W2_2, b2):
    row = edge_index[0]
    col = edge_index[1]
    ew = jnp.ones(row.shape[0], dtype=jnp.float32)
    deg = jnp.zeros((N,), dtype=jnp.float32).at[col].add(ew)
    dis = jnp.where(deg > 0, 1.0 / jnp.sqrt(deg), 0.0)
    norm = dis[row] * ew * dis[col]
    h = jax.nn.relu(_tag_conv(x, row, col, norm, [W1_0, W1_1, W1_2], b1, N))
    out = _tag_conv(h, row, col, norm, [W2_0, W2_1, W2_2], b2, N)
    B = 2000
    return pl.pallas_call(
        _lsm_body,
        out_shape=jax.ShapeDtypeStruct((N, 2), jnp.float32),
        grid=(N // B,),
        in_specs=[pl.BlockSpec((B, 2), lambda i: (i, 0))],
        out_specs=pl.BlockSpec((B, 2), lambda i: (i, 0)),
    )(out)


# speedup vs baseline: 1.6535x; 1.6535x over previous
import functools

import jax
import jax.numpy as jnp
from jax import lax
from jax.experimental import pallas as pl
from jax.experimental.pallas import tpu as pltpu
from jax.experimental.pallas import tpu_sc as plsc

N = 100000
E = 3200000
D_IN = 11
D_HID = 128

SB = 512
NB = 196
NP = NB * SB
NW = 32
NROUND = 7
CH = 128
PADE = 2 * CH
BIG = 3.4028235e38
THR = 1e38

_SC_PARAMS = pltpu.CompilerParams(
    needs_layout_passes=False, use_tc_tiling_on_sc=False)


def _sc_mesh():
    return plsc.VectorSubcoreMesh(core_axis_name="c", subcore_axis_name="s")


def _extract(vec_ref, i):
    return jnp.max(plsc.load_gather(vec_ref, [jnp.full((16,), i, jnp.int32)]))


def _hop_body(dp, hs_hbm, rowi_hbm, coli_hbm, bounds_hbm, out_hbm,
              acc, idxv, colv, relv, rows, bvm, sem):
    nf = dp // 16
    wid = lax.axis_index("s") * 2 + lax.axis_index("c")
    pltpu.sync_copy(bounds_hbm, bvm)

    def bucket_body(r, carry0):
        b = wid + NW * r

        @pl.when(b < NB)
        def _():
            base = b * SB
            lo = _extract(bvm, 2 * b)
            nch = _extract(bvm, 2 * b + 1)

            def init_row(rr, c):
                for f in range(nf):
                    acc[rr, pl.ds(f * 16, 16)] = jnp.full((16,), BIG, jnp.float32)
                return c
            lax.fori_loop(0, SB + 1, init_row, 0)

            def chunk_body(c, carry):
                off = pl.multiple_of(lo + c * CH, 8)
                pltpu.sync_copy(rowi_hbm.at[pl.ds(off, CH)], idxv)
                pltpu.sync_copy(coli_hbm.at[pl.ds(off, CH)], colv)
                pltpu.async_copy(hs_hbm.at[idxv], rows, sem).wait()
                for j in range(CH // 16):
                    cv = colv[pl.ds(j * 16, 16)]
                    ok = (cv >= base) & (cv < base + SB)
                    relv[pl.ds(j * 16, 16)] = jnp.where(ok, cv - base, SB)
                for j in range(CH // 16):
                    rv = relv[pl.ds(j * 16, 16)]
                    for l in range(16):
                        rel = rv[l]
                        e = j * 16 + l
                        for f in range(nf):
                            sl = pl.ds(f * 16, 16)
                            acc[rel, sl] = jnp.minimum(acc[rel, sl], rows[e, sl])
                return carry
            lax.fori_loop(0, nch, chunk_body, 0)
            pltpu.sync_copy(acc.at[pl.ds(0, SB)], out_hbm.at[pl.ds(base, SB)])
        return carry0

    lax.fori_loop(0, NROUND, bucket_body, 0)


def _make_hop(dp):
    return functools.partial(
        pl.kernel,
        functools.partial(_hop_body, dp),
        mesh=_sc_mesh(),
        compiler_params=_SC_PARAMS,
        out_type=jax.ShapeDtypeStruct((NP, dp), jnp.float32),
        scratch_types=[
            pltpu.VMEM((SB + 1, dp), jnp.float32),
            pltpu.VMEM((CH,), jnp.int32),
            pltpu.VMEM((CH,), jnp.int32),
            pltpu.VMEM((CH,), jnp.int32),
            pltpu.VMEM((CH, dp), jnp.float32),
            pltpu.VMEM((512,), jnp.int32),
            pltpu.SemaphoreType.DMA,
        ],
    )()


_hop16 = _make_hop(16)
_hop128 = _make_hop(128)


BN = 2000
BP = 2048


def _scalepad_body(x_ref, dis_ref, o_ref):
    xs = x_ref[...] * dis_ref[...]
    o_ref[...] = jnp.pad(xs, ((0, 0), (0, 16 - D_IN)))


def _scalepad(x, disN):
    return pl.pallas_call(
        _scalepad_body,
        out_shape=jax.ShapeDtypeStruct((N, 16), jnp.float32),
        grid=(N // BN,),
        in_specs=[pl.BlockSpec((BN, D_IN), lambda i: (i, 0)),
                  pl.BlockSpec((BN, 1), lambda i: (i, 0))],
        out_specs=pl.BlockSpec((BN, 16), lambda i: (i, 0)),
    )(x, disN)


def _posthop_body(raw_ref, dis_ref, h_ref, hs_ref):
    raw = raw_ref[...]
    dis = dis_ref[...]
    h = jnp.where(raw >= THR, 0.0, dis * raw)
    h_ref[...] = h
    hs_ref[...] = dis * h


def _posthop(raw, disP, dp):
    return pl.pallas_call(
        _posthop_body,
        out_shape=(jax.ShapeDtypeStruct((NP, dp), jnp.float32),
                   jax.ShapeDtypeStruct((NP, dp), jnp.float32)),
        grid=(NP // BP,),
        in_specs=[pl.BlockSpec((BP, dp), lambda i: (i, 0)),
                  pl.BlockSpec((BP, 1), lambda i: (i, 0))],
        out_specs=(pl.BlockSpec((BP, dp), lambda i: (i, 0)),
                   pl.BlockSpec((BP, dp), lambda i: (i, 0))),
    )(raw, disP)


def _layer1_body(x_ref, h1_ref, h2_ref, w0_ref, w1_ref, w2_ref, b_ref,
                 dis_ref, h_ref, hs_ref):
    o = (x_ref[...] @ w0_ref[...]
         + h1_ref[...][:, :D_IN] @ w1_ref[...]
         + h2_ref[...][:, :D_IN] @ w2_ref[...]
         + b_ref[...])
    h = jnp.maximum(o, 0.0)
    h_ref[...] = h
    hs_ref[...] = h * dis_ref[...]


def _layer1(x, h1p, h2p, W0, W1, W2, b, disN):
    return pl.pallas_call(
        _layer1_body,
        out_shape=(jax.ShapeDtypeStruct((N, D_HID), jnp.float32),
                   jax.ShapeDtypeStruct((N, D_HID), jnp.float32)),
        grid=(N // BN,),
        in_specs=[pl.BlockSpec((BN, D_IN), lambda i: (i, 0)),
                  pl.BlockSpec((BN, 16), lambda i: (i, 0)),
                  pl.BlockSpec((BN, 16), lambda i: (i, 0)),
                  pl.BlockSpec((D_IN, D_HID), lambda i: (0, 0)),
                  pl.BlockSpec((D_IN, D_HID), lambda i: (0, 0)),
                  pl.BlockSpec((D_IN, D_HID), lambda i: (0, 0)),
                  pl.BlockSpec((1, D_HID), lambda i: (0, 0)),
                  pl.BlockSpec((BN, 1), lambda i: (i, 0))],
        out_specs=(pl.BlockSpec((BN, D_HID), lambda i: (i, 0)),
                   pl.BlockSpec((BN, D_HID), lambda i: (i, 0))),
    )(x, h1p, h2p, W0, W1, W2, b, disN)


def _layer2_body(h_ref, g1_ref, g2_ref, w0_ref, w1_ref, w2_ref, b_ref, o_ref):
    z = (h_ref[...] @ w0_ref[...]
         + g1_ref[...] @ w1_ref[...]
         + g2_ref[...] @ w2_ref[...]
         + b_ref[...])
    m = jnp.max(z, axis=1, keepdims=True)
    zz = z - m
    o_ref[...] = zz - jnp.log(jnp.sum(jnp.exp(zz), axis=1, keepdims=True))


def _layer2(h, g1p, g2p, W0, W1, W2, b):
    return pl.pallas_call(
        _layer2_body,
        out_shape=jax.ShapeDtypeStruct((N, 2), jnp.float32),
        grid=(N // BN,),
        in_specs=[pl.BlockSpec((BN, D_HID), lambda i: (i, 0)),
                  pl.BlockSpec((BN, D_HID), lambda i: (i, 0)),
                  pl.BlockSpec((BN, D_HID), lambda i: (i, 0)),
                  pl.BlockSpec((D_HID, 2), lambda i: (0, 0)),
                  pl.BlockSpec((D_HID, 2), lambda i: (0, 0)),
                  pl.BlockSpec((D_HID, 2), lambda i: (0, 0)),
                  pl.BlockSpec((1, 2), lambda i: (0, 0))],
        out_specs=pl.BlockSpec((BN, 2), lambda i: (i, 0)),
    )(h, g1p, g2p, W0, W1, W2, b)


def kernel(x, edge_index, W1_0, W1_1, W1_2, b1, W2_0, W2_1, W2_2, b2):
    row = edge_index[0].astype(jnp.int32)
    col = edge_index[1].astype(jnp.int32)

    order = jnp.argsort(col)
    row_s = row[order]
    col_s = col[order]
    row_p = jnp.concatenate([row_s, jnp.zeros((PADE,), jnp.int32)])
    col_p = jnp.concatenate([col_s, jnp.full((PADE,), -(2 ** 30), jnp.int32)])

    starts = jnp.searchsorted(col_s, jnp.arange(NB + 1, dtype=jnp.int32) * SB)
    starts = starts.astype(jnp.int32)
    lo8 = (starts[:-1] // 8) * 8
    nch = -(-(starts[1:] - lo8) // CH)
    bounds = jnp.zeros((512,), jnp.int32)
    bounds = bounds.at[0:2 * NB:2].set(lo8)
    bounds = bounds.at[1:2 * NB:2].set(nch.astype(jnp.int32))

    node_starts = jnp.searchsorted(col_s, jnp.arange(N + 1, dtype=jnp.int32))
    deg = (node_starts[1:] - node_starts[:-1]).astype(jnp.float32)
    dis = jnp.where(deg > 0, lax.rsqrt(deg), 0.0)
    disN = dis[:, None]
    disP = jnp.pad(disN, ((0, NP - N), (0, 0)))

    xs = _scalepad(x, disN)
    raw1 = _hop16(xs, row_p, col_p, bounds)
    h1p, hs1 = _posthop(raw1, disP, 16)
    raw2 = _hop16(hs1, row_p, col_p, bounds)
    h2p, _ = _posthop(raw2, disP, 16)
    h, hs = _layer1(x, h1p[:N], h2p[:N], W1_0, W1_1, W1_2, b1[None, :], disN)

    raw3 = _hop128(hs, row_p, col_p, bounds)
    g1p, gs1 = _posthop(raw3, disP, D_HID)
    raw4 = _hop128(gs1, row_p, col_p, bounds)
    g2p, _ = _posthop(raw4, disP, D_HID)
    return _layer2(h, g1p[:N], g2p[:N], W2_0, W2_1, W2_2, b2[None, :])


# speedup vs baseline: 5.5211x; 3.3390x over previous
import functools

import jax
import jax.numpy as jnp
from jax import lax
from jax.experimental import pallas as pl
from jax.experimental.pallas import tpu as pltpu
from jax.experimental.pallas import tpu_sc as plsc

N = 100000
E = 3200000
D_IN = 11
D_HID = 128

SB = 512
NB = 196
NP = NB * SB
NW = 32
NROUND = 7
CH = 128
BIG = 3.4028235e38
THR = 1e38

EC1 = 3203072
EPT = EC1 // NW
PADCOL = 100800
NN = 100864
SLC = 6304
CH2 = 256
EP = 3261440

_SC_PARAMS = pltpu.CompilerParams(
    needs_layout_passes=False, use_tc_tiling_on_sc=False)


def _sc_mesh():
    return plsc.VectorSubcoreMesh(core_axis_name="c", subcore_axis_name="s")


def _extract(vec_ref, i):
    return jnp.max(plsc.load_gather(vec_ref, [jnp.full((16,), i, jnp.int32)]))


def _c1_body(col_hbm, deg_hbm, cnt_hbm, shared, shhist, zbuf, colv, bvv,
             ones, hrow):
    cid = lax.axis_index("c")
    sid = lax.axis_index("s")
    wid = sid * 2 + cid

    def zrow(i, c):
        zbuf[pl.ds(i * 16, 16)] = jnp.zeros((16,), jnp.float32)
        return c
    lax.fori_loop(0, (SLC // 2) // 16, zrow, 0)
    for i in range(CH // 16):
        ones[pl.ds(i * 16, 16)] = jnp.ones((16,), jnp.float32)
    pltpu.sync_copy(zbuf, shared.at[pl.ds(sid * SLC, SLC // 2)])
    pltpu.sync_copy(zbuf, shared.at[pl.ds(sid * SLC + SLC // 2, SLC // 2)])
    pltpu.sync_copy(zbuf.at[pl.ds(0, 256)], shhist.at[pl.ds(sid * 256, 256)])
    plsc.subcore_barrier()

    def chunk(c, carry):
        off = pl.multiple_of(wid * EPT + c * CH, 8)
        pltpu.sync_copy(col_hbm.at[pl.ds(off, CH)], colv)
        for j in range(CH // 16):
            bvv[pl.ds(j * 16, 16)] = (colv[pl.ds(j * 16, 16)] >> 9) + sid * 256
        pltpu.sync_copy(ones, shared.at[colv], add=True)
        pltpu.sync_copy(ones, shhist.at[bvv], add=True)
        return carry
    lax.fori_loop(0, EPT // CH, chunk, 0)
    plsc.subcore_barrier()

    pltpu.sync_copy(shhist.at[pl.ds(sid * 256, 256)], hrow)
    pltpu.sync_copy(hrow, cnt_hbm.at[wid])
    pltpu.sync_copy(shared.at[pl.ds(sid * SLC, SLC // 2)],
                    deg_hbm.at[cid, pl.ds(sid * SLC, SLC // 2)])
    pltpu.sync_copy(shared.at[pl.ds(sid * SLC + SLC // 2, SLC // 2)],
                    deg_hbm.at[cid, pl.ds(sid * SLC + SLC // 2, SLC // 2)])


_c1 = functools.partial(
    pl.kernel, _c1_body, mesh=_sc_mesh(), compiler_params=_SC_PARAMS,
    out_type=(jax.ShapeDtypeStruct((2, NN), jnp.float32),
              jax.ShapeDtypeStruct((NW, 256), jnp.float32)),
    scratch_types=[
        pltpu.VMEM_SHARED((16 * SLC,), jnp.float32),
        pltpu.VMEM_SHARED((16 * 256,), jnp.float32),
        pltpu.VMEM((SLC // 2,), jnp.float32),
        pltpu.VMEM((CH,), jnp.int32),
        pltpu.VMEM((CH,), jnp.int32),
        pltpu.VMEM((CH,), jnp.float32),
        pltpu.VMEM((256,), jnp.float32),
    ],
)()


def _c2_body(row_hbm, col_hbm, offs_hbm, orow_hbm, ocol_hbm,
             stag_r, stag_c, ap, wr, offv, rowv, colv):
    cid = lax.axis_index("c")
    sid = lax.axis_index("s")
    wid = sid * 2 + cid
    pltpu.sync_copy(offs_hbm.at[wid], offv)

    def initb(b, c):
        ap[b, pl.ds(0, 16)] = jnp.zeros((16,), jnp.int32)
        wr[b, pl.ds(0, 16)] = plsc.load_gather(
            offv, [jnp.full((16,), b, jnp.int32)])
        return c
    lax.fori_loop(0, 256, initb, 0)

    lane_iota = lax.iota(jnp.int32, 16)

    def chunk(c, carry):
        off = pl.multiple_of(wid * EPT + c * CH2, 8)
        pltpu.sync_copy(row_hbm.at[pl.ds(off, CH2)], rowv)
        pltpu.sync_copy(col_hbm.at[pl.ds(off, CH2)], colv)
        for j in range(CH2 // 16):
            cv = colv[pl.ds(j * 16, 16)]
            rv = rowv[pl.ds(j * 16, 16)]
            bv = cv >> 9
            for l in range(16):
                b = bv[l]
                apv = ap[b, :]
                a0 = apv[0]
                bidx = jnp.full((16,), b, jnp.int32)
                aidx = jnp.full((16,), a0, jnp.int32)
                lmask = lane_iota == l
                plsc.store_scatter(stag_r, [bidx, aidx], rv, mask=lmask)
                plsc.store_scatter(stag_c, [bidx, aidx], cv, mask=lmask)
                ap[b, :] = apv + 1

                @pl.when(a0 + 1 >= 32)
                def _():
                    wv = wr[b, :]
                    w0 = pl.multiple_of(wv[0], 8)
                    pltpu.sync_copy(stag_r.at[b], orow_hbm.at[pl.ds(w0, 32)])
                    pltpu.sync_copy(stag_c.at[b], ocol_hbm.at[pl.ds(w0, 32)])
                    wr[b, :] = wv + 32
                    ap[b, :] = jnp.zeros((16,), jnp.int32)
        return carry
    lax.fori_loop(0, EPT // CH2, chunk, 0)

    def tailb(b, c):
        apv = ap[b, :]
        a0 = apv[0]

        @pl.when(a0 > 0)
        def _():
            def padi(i, c2):
                sl = pl.ds(i * 16, 16)
                pos = lane_iota + i * 16
                padm = pos >= a0
                stag_c[b, sl] = jnp.where(padm, PADCOL, stag_c[b, sl])
                stag_r[b, sl] = jnp.where(padm, 0, stag_r[b, sl])
                return c2
            lax.fori_loop(0, 2, padi, 0)
            wv = wr[b, :]
            for kk in range(4):
                @pl.when(a0 > 8 * kk)
                def _():
                    w0 = pl.multiple_of(wv[0] + 8 * kk, 8)
                    pltpu.sync_copy(stag_r.at[b, pl.ds(8 * kk, 8)],
                                    orow_hbm.at[pl.ds(w0, 8)])
                    pltpu.sync_copy(stag_c.at[b, pl.ds(8 * kk, 8)],
                                    ocol_hbm.at[pl.ds(w0, 8)])
        return c
    lax.fori_loop(0, 256, tailb, 0)


_c2 = functools.partial(
    pl.kernel, _c2_body, mesh=_sc_mesh(), compiler_params=_SC_PARAMS,
    out_type=(jax.ShapeDtypeStruct((EP,), jnp.int32),
              jax.ShapeDtypeStruct((EP,), jnp.int32)),
    scratch_types=[
        pltpu.VMEM((256, 32), jnp.int32),
        pltpu.VMEM((256, 32), jnp.int32),
        pltpu.VMEM((256, 16), jnp.int32),
        pltpu.VMEM((256, 16), jnp.int32),
        pltpu.VMEM((256,), jnp.int32),
        pltpu.VMEM((CH2,), jnp.int32),
        pltpu.VMEM((CH2,), jnp.int32),
    ],
)()


def _hop_body(dp, hs_hbm, rowi_hbm, coli_hbm, bounds_hbm, out_hbm,
              acc, idxv, colv, relv, rows, bvm, sem):
    nf = dp // 16
    wid = lax.axis_index("s") * 2 + lax.axis_index("c")
    pltpu.sync_copy(bounds_hbm, bvm)

    def bucket_body(r, carry0):
        b = wid + NW * r

        @pl.when(b < NB)
        def _():
            base = b * SB
            lo = _extract(bvm, 2 * b)
            nch = _extract(bvm, 2 * b + 1)

            def init_row(rr, c):
                for f in range(nf):
                    acc[rr, pl.ds(f * 16, 16)] = jnp.full((16,), BIG, jnp.float32)
                return c
            lax.fori_loop(0, SB + 1, init_row, 0)

            def chunk_body(c, carry):
                off = pl.multiple_of(lo + c * CH, 8)
                pltpu.sync_copy(rowi_hbm.at[pl.ds(off, CH)], idxv)
                pltpu.sync_copy(coli_hbm.at[pl.ds(off, CH)], colv)
                pltpu.async_copy(hs_hbm.at[idxv], rows, sem).wait()
                for j in range(CH // 16):
                    cv = colv[pl.ds(j * 16, 16)]
                    ok = (cv >= base) & (cv < base + SB)
                    relv[pl.ds(j * 16, 16)] = jnp.where(ok, cv - base, SB)
                for j in range(CH // 16):
                    rv = relv[pl.ds(j * 16, 16)]
                    for l in range(16):
                        rel = rv[l]
                        e = j * 16 + l
                        for f in range(nf):
                            sl = pl.ds(f * 16, 16)
                            acc[rel, sl] = jnp.minimum(acc[rel, sl], rows[e, sl])
                return carry
            lax.fori_loop(0, nch, chunk_body, 0)
            pltpu.sync_copy(acc.at[pl.ds(0, SB)], out_hbm.at[pl.ds(base, SB)])
        return carry0

    lax.fori_loop(0, NROUND, bucket_body, 0)


def _make_hop(dp):
    return functools.partial(
        pl.kernel,
        functools.partial(_hop_body, dp),
        mesh=_sc_mesh(),
        compiler_params=_SC_PARAMS,
        out_type=jax.ShapeDtypeStruct((NP, dp), jnp.float32),
        scratch_types=[
            pltpu.VMEM((SB + 1, dp), jnp.float32),
            pltpu.VMEM((CH,), jnp.int32),
            pltpu.VMEM((CH,), jnp.int32),
            pltpu.VMEM((CH,), jnp.int32),
            pltpu.VMEM((CH, dp), jnp.float32),
            pltpu.VMEM((512,), jnp.int32),
            pltpu.SemaphoreType.DMA,
        ],
    )()


_hop16 = _make_hop(16)
_hop128 = _make_hop(128)


BN = 2000
BP = 2048


def _scalepad_body(x_ref, deg_ref, o_ref, dis_ref):
    deg = deg_ref[:, 0] + deg_ref[:, 1]
    dis = jnp.where(deg > 0, lax.rsqrt(deg), 0.0)[:, None]
    dis_ref[...] = dis
    o_ref[...] = jnp.pad(x_ref[...] * dis, ((0, 0), (0, 16 - D_IN)))


def _scalepad(x, deg2):
    return pl.pallas_call(
        _scalepad_body,
        out_shape=(jax.ShapeDtypeStruct((N, 16), jnp.float32),
                   jax.ShapeDtypeStruct((N, 1), jnp.float32)),
        grid=(N // BN,),
        in_specs=[pl.BlockSpec((BN, D_IN), lambda i: (i, 0)),
                  pl.BlockSpec((BN, 2), lambda i: (i, 0))],
        out_specs=(pl.BlockSpec((BN, 16), lambda i: (i, 0)),
                   pl.BlockSpec((BN, 1), lambda i: (i, 0))),
    )(x, deg2)


def _posthop_body(raw_ref, dis_ref, h_ref, hs_ref):
    raw = raw_ref[...]
    dis = dis_ref[...]
    h = jnp.where(raw >= THR, 0.0, dis * raw)
    h_ref[...] = h
    hs_ref[...] = dis * h


def _posthop(raw, disP, dp):
    return pl.pallas_call(
        _posthop_body,
        out_shape=(jax.ShapeDtypeStruct((NP, dp), jnp.float32),
                   jax.ShapeDtypeStruct((NP, dp), jnp.float32)),
        grid=(NP // BP,),
        in_specs=[pl.BlockSpec((BP, dp), lambda i: (i, 0)),
                  pl.BlockSpec((BP, 1), lambda i: (i, 0))],
        out_specs=(pl.BlockSpec((BP, dp), lambda i: (i, 0)),
                   pl.BlockSpec((BP, dp), lambda i: (i, 0))),
    )(raw, disP)


def _layer1_body(x_ref, h1_ref, h2_ref, w0_ref, w1_ref, w2_ref, b_ref,
                 dis_ref, h_ref, hs_ref):
    o = (x_ref[...] @ w0_ref[...]
         + h1_ref[...][:, :D_IN] @ w1_ref[...]
         + h2_ref[...][:, :D_IN] @ w2_ref[...]
         + b_ref[...])
    h = jnp.maximum(o, 0.0)
    h_ref[...] = h
    hs_ref[...] = h * dis_ref[...]


def _layer1(x, h1p, h2p, W0, W1, W2, b, disN):
    return pl.pallas_call(
        _layer1_body,
        out_shape=(jax.ShapeDtypeStruct((N, D_HID), jnp.float32),
                   jax.ShapeDtypeStruct((N, D_HID), jnp.float32)),
        grid=(N // BN,),
        in_specs=[pl.BlockSpec((BN, D_IN), lambda i: (i, 0)),
                  pl.BlockSpec((BN, 16), lambda i: (i, 0)),
                  pl.BlockSpec((BN, 16), lambda i: (i, 0)),
                  pl.BlockSpec((D_IN, D_HID), lambda i: (0, 0)),
                  pl.BlockSpec((D_IN, D_HID), lambda i: (0, 0)),
                  pl.BlockSpec((D_IN, D_HID), lambda i: (0, 0)),
                  pl.BlockSpec((1, D_HID), lambda i: (0, 0)),
                  pl.BlockSpec((BN, 1), lambda i: (i, 0))],
        out_specs=(pl.BlockSpec((BN, D_HID), lambda i: (i, 0)),
                   pl.BlockSpec((BN, D_HID), lambda i: (i, 0))),
    )(x, h1p, h2p, W0, W1, W2, b, disN)


def _layer2_body(h_ref, g1_ref, g2_ref, w0_ref, w1_ref, w2_ref, b_ref, o_ref):
    z = (h_ref[...] @ w0_ref[...]
         + g1_ref[...] @ w1_ref[...]
         + g2_ref[...] @ w2_ref[...]
         + b_ref[...])
    m = jnp.max(z, axis=1, keepdims=True)
    zz = z - m
    o_ref[...] = zz - jnp.log(jnp.sum(jnp.exp(zz), axis=1, keepdims=True))


def _layer2(h, g1p, g2p, W0, W1, W2, b):
    return pl.pallas_call(
        _layer2_body,
        out_shape=jax.ShapeDtypeStruct((N, 2), jnp.float32),
        grid=(N // BN,),
        in_specs=[pl.BlockSpec((BN, D_HID), lambda i: (i, 0)),
                  pl.BlockSpec((BN, D_HID), lambda i: (i, 0)),
                  pl.BlockSpec((BN, D_HID), lambda i: (i, 0)),
                  pl.BlockSpec((D_HID, 2), lambda i: (0, 0)),
                  pl.BlockSpec((D_HID, 2), lambda i: (0, 0)),
                  pl.BlockSpec((D_HID, 2), lambda i: (0, 0)),
                  pl.BlockSpec((1, 2), lambda i: (0, 0))],
        out_specs=pl.BlockSpec((BN, 2), lambda i: (i, 0)),
    )(h, g1p, g2p, W0, W1, W2, b)


def kernel(x, edge_index, W1_0, W1_1, W1_2, b1, W2_0, W2_1, W2_2, b2):
    row = edge_index[0].astype(jnp.int32)
    col = edge_index[1].astype(jnp.int32)
    row_p = jnp.concatenate([row, jnp.zeros((EC1 - E,), jnp.int32)])
    col_p = jnp.concatenate([col, jnp.full((EC1 - E,), PADCOL, jnp.int32)])

    deg2, cnt_f = _c1(col_p)

    cnt = cnt_f.astype(jnp.int32)
    seg8 = (cnt + 7) // 8 * 8
    bt8 = seg8.sum(axis=0)
    base = jnp.concatenate([jnp.zeros((1,), jnp.int32),
                            jnp.cumsum(bt8)[:-1].astype(jnp.int32)])
    seg_off = base[None, :] + (jnp.cumsum(seg8, axis=0) - seg8)
    bounds = jnp.zeros((512,), jnp.int32)
    bounds = bounds.at[0:2 * NB:2].set(base[:NB])
    bounds = bounds.at[1:2 * NB:2].set(-(-bt8[:NB] // CH))

    row_b, col_b = _c2(row_p, col_p, seg_off)

    xs, disN = _scalepad(x, deg2[:, :N].T)
    disP = jnp.pad(disN, ((0, NP - N), (0, 0)))
    raw1 = _hop16(xs, row_b, col_b, bounds)
    h1p, hs1 = _posthop(raw1, disP, 16)
    raw2 = _hop16(hs1, row_b, col_b, bounds)
    h2p, _ = _posthop(raw2, disP, 16)
    h, hs = _layer1(x, h1p[:N], h2p[:N], W1_0, W1_1, W1_2, b1[None, :], disN)

    raw3 = _hop128(hs, row_b, col_b, bounds)
    g1p, gs1 = _posthop(raw3, disP, D_HID)
    raw4 = _hop128(gs1, row_b, col_b, bounds)
    g2p, _ = _posthop(raw4, disP, D_HID)
    return _layer2(h, g1p[:N], g2p[:N], W2_0, W2_1, W2_2, b2[None, :])


# speedup vs baseline: 7.2753x; 1.3177x over previous
import functools

import jax
import jax.numpy as jnp
from jax import lax
from jax.experimental import pallas as pl
from jax.experimental.pallas import tpu as pltpu
from jax.experimental.pallas import tpu_sc as plsc

N = 100000
E = 3200000
D_IN = 11
D_HID = 128

SB = 512
NB = 196
NP = NB * SB
NW = 32
NROUND = 7
CH = 128
BIG = 3.4028235e38
THR = 1e38

EC1 = 3203072
EPT = EC1 // NW
PADCOL = 100800
NN = 100864
SLC = 6304
CH2 = 256
EP = 3261440

_SC_PARAMS = pltpu.CompilerParams(
    needs_layout_passes=False, use_tc_tiling_on_sc=False)


def _sc_mesh():
    return plsc.VectorSubcoreMesh(core_axis_name="c", subcore_axis_name="s")


def _extract(vec_ref, i):
    return jnp.max(plsc.load_gather(vec_ref, [jnp.full((16,), i, jnp.int32)]))


def _c1_body(col_hbm, deg_hbm, cnt_hbm, shared, shhist, zbuf, colv, bvv,
             ones, hrow):
    cid = lax.axis_index("c")
    sid = lax.axis_index("s")
    wid = sid * 2 + cid

    def zrow(i, c):
        zbuf[pl.ds(i * 16, 16)] = jnp.zeros((16,), jnp.float32)
        return c
    lax.fori_loop(0, (SLC // 2) // 16, zrow, 0)
    for i in range(CH // 16):
        ones[pl.ds(i * 16, 16)] = jnp.ones((16,), jnp.float32)
    pltpu.sync_copy(zbuf, shared.at[pl.ds(sid * SLC, SLC // 2)])
    pltpu.sync_copy(zbuf, shared.at[pl.ds(sid * SLC + SLC // 2, SLC // 2)])
    pltpu.sync_copy(zbuf.at[pl.ds(0, 256)], shhist.at[pl.ds(sid * 256, 256)])
    plsc.subcore_barrier()

    def chunk(c, carry):
        off = pl.multiple_of(wid * EPT + c * CH, 8)
        pltpu.sync_copy(col_hbm.at[pl.ds(off, CH)], colv)
        for j in range(CH // 16):
            bvv[pl.ds(j * 16, 16)] = (colv[pl.ds(j * 16, 16)] >> 9) + sid * 256
        pltpu.sync_copy(ones, shared.at[colv], add=True)
        pltpu.sync_copy(ones, shhist.at[bvv], add=True)
        return carry
    lax.fori_loop(0, EPT // CH, chunk, 0)
    plsc.subcore_barrier()

    pltpu.sync_copy(shhist.at[pl.ds(sid * 256, 256)], hrow)
    pltpu.sync_copy(hrow, cnt_hbm.at[wid])
    pltpu.sync_copy(shared.at[pl.ds(sid * SLC, SLC // 2)],
                    deg_hbm.at[cid, pl.ds(sid * SLC, SLC // 2)])
    pltpu.sync_copy(shared.at[pl.ds(sid * SLC + SLC // 2, SLC // 2)],
                    deg_hbm.at[cid, pl.ds(sid * SLC + SLC // 2, SLC // 2)])


_c1 = functools.partial(
    pl.kernel, _c1_body, mesh=_sc_mesh(), compiler_params=_SC_PARAMS,
    out_type=(jax.ShapeDtypeStruct((2, NN), jnp.float32),
              jax.ShapeDtypeStruct((NW, 256), jnp.float32)),
    scratch_types=[
        pltpu.VMEM_SHARED((16 * SLC,), jnp.float32),
        pltpu.VMEM_SHARED((16 * 256,), jnp.float32),
        pltpu.VMEM((SLC // 2,), jnp.float32),
        pltpu.VMEM((CH,), jnp.int32),
        pltpu.VMEM((CH,), jnp.int32),
        pltpu.VMEM((CH,), jnp.float32),
        pltpu.VMEM((256,), jnp.float32),
    ],
)()


def _c2_body(row_hbm, col_hbm, offs_hbm, orow_hbm, ocol_hbm,
             stag_r, stag_c, ap, wr, offv, rowv, colv):
    cid = lax.axis_index("c")
    sid = lax.axis_index("s")
    wid = sid * 2 + cid
    pltpu.sync_copy(offs_hbm.at[wid], offv)

    def initb(b, c):
        ap[b, pl.ds(0, 16)] = jnp.zeros((16,), jnp.int32)
        wr[b, pl.ds(0, 16)] = plsc.load_gather(
            offv, [jnp.full((16,), b, jnp.int32)])
        return c
    lax.fori_loop(0, 256, initb, 0)

    lane_iota = lax.iota(jnp.int32, 16)

    def chunk(c, carry):
        off = pl.multiple_of(wid * EPT + c * CH2, 8)
        pltpu.sync_copy(row_hbm.at[pl.ds(off, CH2)], rowv)
        pltpu.sync_copy(col_hbm.at[pl.ds(off, CH2)], colv)
        for j in range(CH2 // 16):
            cv = colv[pl.ds(j * 16, 16)]
            rv = rowv[pl.ds(j * 16, 16)]
            bv = cv >> 9
            for l in range(16):
                b = bv[l]
                apv = ap[b, :]
                a0 = apv[0]
                bidx = jnp.full((16,), b, jnp.int32)
                aidx = jnp.full((16,), a0, jnp.int32)
                lmask = lane_iota == l
                plsc.store_scatter(stag_r, [bidx, aidx], rv, mask=lmask)
                plsc.store_scatter(stag_c, [bidx, aidx], cv, mask=lmask)
                ap[b, :] = apv + 1

                @pl.when(a0 + 1 >= 32)
                def _():
                    wv = wr[b, :]
                    w0 = pl.multiple_of(wv[0], 8)
                    pltpu.sync_copy(stag_r.at[b], orow_hbm.at[pl.ds(w0, 32)])
                    pltpu.sync_copy(stag_c.at[b], ocol_hbm.at[pl.ds(w0, 32)])
                    wr[b, :] = wv + 32
                    ap[b, :] = jnp.zeros((16,), jnp.int32)
        return carry
    lax.fori_loop(0, EPT // CH2, chunk, 0)

    def tailb(b, c):
        apv = ap[b, :]
        a0 = apv[0]

        @pl.when(a0 > 0)
        def _():
            def padi(i, c2):
                sl = pl.ds(i * 16, 16)
                pos = lane_iota + i * 16
                padm = pos >= a0
                stag_c[b, sl] = jnp.where(padm, PADCOL, stag_c[b, sl])
                stag_r[b, sl] = jnp.where(padm, 0, stag_r[b, sl])
                return c2
            lax.fori_loop(0, 2, padi, 0)
            wv = wr[b, :]
            for kk in range(4):
                @pl.when(a0 > 8 * kk)
                def _():
                    w0 = pl.multiple_of(wv[0] + 8 * kk, 8)
                    pltpu.sync_copy(stag_r.at[b, pl.ds(8 * kk, 8)],
                                    orow_hbm.at[pl.ds(w0, 8)])
                    pltpu.sync_copy(stag_c.at[b, pl.ds(8 * kk, 8)],
                                    ocol_hbm.at[pl.ds(w0, 8)])
        return c
    lax.fori_loop(0, 256, tailb, 0)


_c2 = functools.partial(
    pl.kernel, _c2_body, mesh=_sc_mesh(), compiler_params=_SC_PARAMS,
    out_type=(jax.ShapeDtypeStruct((EP,), jnp.int32),
              jax.ShapeDtypeStruct((EP,), jnp.int32)),
    scratch_types=[
        pltpu.VMEM((256, 32), jnp.int32),
        pltpu.VMEM((256, 32), jnp.int32),
        pltpu.VMEM((256, 16), jnp.int32),
        pltpu.VMEM((256, 16), jnp.int32),
        pltpu.VMEM((256,), jnp.int32),
        pltpu.VMEM((CH2,), jnp.int32),
        pltpu.VMEM((CH2,), jnp.int32),
    ],
)()


def _hop_body(dp, hs_hbm, rowi_hbm, coli_hbm, bounds_hbm, out_hbm,
              acc, idxv, colv, relv, rows, bvm, semi, semg):
    nf = dp // 16
    wid = lax.axis_index("s") * 2 + lax.axis_index("c")
    pltpu.sync_copy(bounds_hbm, bvm)

    def issue_idx(lo, c):
        off = pl.multiple_of(lo + c * CH, 8)
        so = (c % 2) * CH
        pltpu.async_copy(rowi_hbm.at[pl.ds(off, CH)],
                         idxv.at[pl.ds(so, CH)], semi)
        pltpu.async_copy(coli_hbm.at[pl.ds(off, CH)],
                         colv.at[pl.ds(so, CH)], semi)

    def wait_idx():
        for _ in range(2):
            pltpu.make_async_copy(rowi_hbm.at[pl.ds(0, CH)],
                                  idxv.at[pl.ds(0, CH)], semi).wait()

    def issue_gather(c):
        so = (c % 2) * CH
        pltpu.async_copy(hs_hbm.at[idxv.at[pl.ds(so, CH)]],
                         rows.at[pl.ds(so, CH)], semg)

    def wait_gather():
        pltpu.make_async_copy(hs_hbm.at[pl.ds(0, CH)],
                              rows.at[pl.ds(0, CH)], semg).wait()

    def bucket_body(r, carry0):
        b = wid + NW * r

        @pl.when(b < NB)
        def _():
            base = b * SB
            lo = _extract(bvm, 2 * b)
            nch = _extract(bvm, 2 * b + 1)

            def init_row(rr, c):
                for f in range(nf):
                    acc[rr, pl.ds(f * 16, 16)] = jnp.full((16,), BIG, jnp.float32)
                return c
            lax.fori_loop(0, SB + 1, init_row, 0)

            @pl.when(nch > 0)
            def _():
                issue_idx(lo, 0)
                wait_idx()
                issue_gather(0)

                @pl.when(nch > 1)
                def _():
                    issue_idx(lo, 1)

                def chunk_body(c, carry):
                    wait_gather()

                    @pl.when(c + 2 < nch)
                    def _():
                        issue_idx(lo, c + 2)

                    @pl.when(c + 1 < nch)
                    def _():
                        wait_idx()
                        issue_gather(c + 1)

                    so = (c % 2) * CH
                    for j in range(CH // 16):
                        cv = colv[pl.ds(so + j * 16, 16)]
                        ok = (cv >= base) & (cv < base + SB)
                        relv[pl.ds(j * 16, 16)] = jnp.where(ok, cv - base, SB)
                    for j in range(CH // 16):
                        rv = relv[pl.ds(j * 16, 16)]
                        for l in range(16):
                            rel = rv[l]
                            e = so + j * 16 + l
                            for f in range(nf):
                                sl = pl.ds(f * 16, 16)
                                acc[rel, sl] = jnp.minimum(acc[rel, sl],
                                                           rows[e, sl])
                    return carry
                lax.fori_loop(0, nch, chunk_body, 0)
            pltpu.sync_copy(acc.at[pl.ds(0, SB)], out_hbm.at[pl.ds(base, SB)])
        return carry0

    lax.fori_loop(0, NROUND, bucket_body, 0)


def _make_hop(dp):
    return functools.partial(
        pl.kernel,
        functools.partial(_hop_body, dp),
        mesh=_sc_mesh(),
        compiler_params=_SC_PARAMS,
        out_type=jax.ShapeDtypeStruct((NP, dp), jnp.float32),
        scratch_types=[
            pltpu.VMEM((SB + 1, dp), jnp.float32),
            pltpu.VMEM((2 * CH,), jnp.int32),
            pltpu.VMEM((2 * CH,), jnp.int32),
            pltpu.VMEM((CH,), jnp.int32),
            pltpu.VMEM((2 * CH, dp), jnp.float32),
            pltpu.VMEM((512,), jnp.int32),
            pltpu.SemaphoreType.DMA,
            pltpu.SemaphoreType.DMA,
        ],
    )()


_hop16 = _make_hop(16)
_hop128 = _make_hop(128)


BN = 2000
BP = 2048


def _scalepad_body(x_ref, deg_ref, o_ref, dis_ref):
    deg = deg_ref[:, 0] + deg_ref[:, 1]
    dis = jnp.where(deg > 0, lax.rsqrt(deg), 0.0)[:, None]
    dis_ref[...] = dis
    o_ref[...] = jnp.pad(x_ref[...] * dis, ((0, 0), (0, 16 - D_IN)))


def _scalepad(x, deg2):
    return pl.pallas_call(
        _scalepad_body,
        out_shape=(jax.ShapeDtypeStruct((N, 16), jnp.float32),
                   jax.ShapeDtypeStruct((N, 1), jnp.float32)),
        grid=(N // BN,),
        in_specs=[pl.BlockSpec((BN, D_IN), lambda i: (i, 0)),
                  pl.BlockSpec((BN, 2), lambda i: (i, 0))],
        out_specs=(pl.BlockSpec((BN, 16), lambda i: (i, 0)),
                   pl.BlockSpec((BN, 1), lambda i: (i, 0))),
    )(x, deg2)


def _posthop_body(raw_ref, dis_ref, h_ref, hs_ref):
    raw = raw_ref[...]
    dis = dis_ref[...]
    h = jnp.where(raw >= THR, 0.0, dis * raw)
    h_ref[...] = h
    hs_ref[...] = dis * h


def _posthop(raw, disP, dp):
    return pl.pallas_call(
        _posthop_body,
        out_shape=(jax.ShapeDtypeStruct((NP, dp), jnp.float32),
                   jax.ShapeDtypeStruct((NP, dp), jnp.float32)),
        grid=(NP // BP,),
        in_specs=[pl.BlockSpec((BP, dp), lambda i: (i, 0)),
                  pl.BlockSpec((BP, 1), lambda i: (i, 0))],
        out_specs=(pl.BlockSpec((BP, dp), lambda i: (i, 0)),
                   pl.BlockSpec((BP, dp), lambda i: (i, 0))),
    )(raw, disP)


def _layer1_body(x_ref, h1_ref, h2_ref, w0_ref, w1_ref, w2_ref, b_ref,
                 dis_ref, h_ref, hs_ref):
    o = (x_ref[...] @ w0_ref[...]
         + h1_ref[...][:, :D_IN] @ w1_ref[...]
         + h2_ref[...][:, :D_IN] @ w2_ref[...]
         + b_ref[...])
    h = jnp.maximum(o, 0.0)
    h_ref[...] = h
    hs_ref[...] = h * dis_ref[...]


def _layer1(x, h1p, h2p, W0, W1, W2, b, disN):
    return pl.pallas_call(
        _layer1_body,
        out_shape=(jax.ShapeDtypeStruct((N, D_HID), jnp.float32),
                   jax.ShapeDtypeStruct((N, D_HID), jnp.float32)),
        grid=(N // BN,),
        in_specs=[pl.BlockSpec((BN, D_IN), lambda i: (i, 0)),
                  pl.BlockSpec((BN, 16), lambda i: (i, 0)),
                  pl.BlockSpec((BN, 16), lambda i: (i, 0)),
                  pl.BlockSpec((D_IN, D_HID), lambda i: (0, 0)),
                  pl.BlockSpec((D_IN, D_HID), lambda i: (0, 0)),
                  pl.BlockSpec((D_IN, D_HID), lambda i: (0, 0)),
                  pl.BlockSpec((1, D_HID), lambda i: (0, 0)),
                  pl.BlockSpec((BN, 1), lambda i: (i, 0))],
        out_specs=(pl.BlockSpec((BN, D_HID), lambda i: (i, 0)),
                   pl.BlockSpec((BN, D_HID), lambda i: (i, 0))),
    )(x, h1p, h2p, W0, W1, W2, b, disN)


def _layer2_body(h_ref, g1_ref, g2_ref, w0_ref, w1_ref, w2_ref, b_ref, o_ref):
    z = (h_ref[...] @ w0_ref[...]
         + g1_ref[...] @ w1_ref[...]
         + g2_ref[...] @ w2_ref[...]
         + b_ref[...])
    m = jnp.max(z, axis=1, keepdims=True)
    zz = z - m
    o_ref[...] = zz - jnp.log(jnp.sum(jnp.exp(zz), axis=1, keepdims=True))


def _layer2(h, g1p, g2p, W0, W1, W2, b):
    return pl.pallas_call(
        _layer2_body,
        out_shape=jax.ShapeDtypeStruct((N, 2), jnp.float32),
        grid=(N // BN,),
        in_specs=[pl.BlockSpec((BN, D_HID), lambda i: (i, 0)),
                  pl.BlockSpec((BN, D_HID), lambda i: (i, 0)),
                  pl.BlockSpec((BN, D_HID), lambda i: (i, 0)),
                  pl.BlockSpec((D_HID, 2), lambda i: (0, 0)),
                  pl.BlockSpec((D_HID, 2), lambda i: (0, 0)),
                  pl.BlockSpec((D_HID, 2), lambda i: (0, 0)),
                  pl.BlockSpec((1, 2), lambda i: (0, 0))],
        out_specs=pl.BlockSpec((BN, 2), lambda i: (i, 0)),
    )(h, g1p, g2p, W0, W1, W2, b)


def kernel(x, edge_index, W1_0, W1_1, W1_2, b1, W2_0, W2_1, W2_2, b2):
    row = edge_index[0].astype(jnp.int32)
    col = edge_index[1].astype(jnp.int32)
    row_p = jnp.concatenate([row, jnp.zeros((EC1 - E,), jnp.int32)])
    col_p = jnp.concatenate([col, jnp.full((EC1 - E,), PADCOL, jnp.int32)])

    deg2, cnt_f = _c1(col_p)

    cnt = cnt_f.astype(jnp.int32)
    seg8 = (cnt + 7) // 8 * 8
    bt8 = seg8.sum(axis=0)
    base = jnp.concatenate([jnp.zeros((1,), jnp.int32),
                            jnp.cumsum(bt8)[:-1].astype(jnp.int32)])
    seg_off = base[None, :] + (jnp.cumsum(seg8, axis=0) - seg8)
    bounds = jnp.zeros((512,), jnp.int32)
    bounds = bounds.at[0:2 * NB:2].set(base[:NB])
    bounds = bounds.at[1:2 * NB:2].set(-(-bt8[:NB] // CH))

    row_b, col_b = _c2(row_p, col_p, seg_off)

    xs, disN = _scalepad(x, deg2[:, :N].T)
    disP = jnp.pad(disN, ((0, NP - N), (0, 0)))
    raw1 = _hop16(xs, row_b, col_b, bounds)
    h1p, hs1 = _posthop(raw1, disP, 16)
    raw2 = _hop16(hs1, row_b, col_b, bounds)
    h2p, _ = _posthop(raw2, disP, 16)
    h, hs = _layer1(x, h1p[:N], h2p[:N], W1_0, W1_1, W1_2, b1[None, :], disN)

    raw3 = _hop128(hs, row_b, col_b, bounds)
    g1p, gs1 = _posthop(raw3, disP, D_HID)
    raw4 = _hop128(gs1, row_b, col_b, bounds)
    g2p, _ = _posthop(raw4, disP, D_HID)
    return _layer2(h, g1p[:N], g2p[:N], W2_0, W2_1, W2_2, b2[None, :])
